# Initial kernel scaffold; baseline (speedup 1.0000x reference)
import functools

import jax
import jax.numpy as jnp
from jax import lax
from jax.experimental import pallas as pl
from jax.experimental.pallas import tpu as pltpu
from jax.experimental.pallas import tpu_sc as plsc

N = 10000
E = 320000
D = 128
NC = 2
NS = 16
NW = NC * NS
EPW = E // NW
CH = 80
NCHUNK = EPW // CH
RPS = N // NS
ZCH = 125
DEGW = 16

_mesh = plsc.VectorSubcoreMesh(core_axis_name="c", subcore_axis_name="s")

_ZERO16 = jnp.zeros((16,), jnp.float32)
_ONE16 = jnp.ones((16,), jnp.float32)


def _ids():
    c = lax.axis_index("c")
    s = lax.axis_index("s")
    return c, s, c * NS + s


@functools.partial(
    pl.kernel,
    out_type=jax.ShapeDtypeStruct((NC * N, DEGW), jnp.float32),
    mesh=_mesh,
    scratch_types=[
        pltpu.VMEM((CH,), jnp.int32),
        pltpu.VMEM((CH, DEGW), jnp.float32),
        pltpu.VMEM((ZCH, DEGW), jnp.float32),
        pltpu.VMEM_SHARED((N, DEGW), jnp.float32),
    ],
)
def _deg_hist(dst_hbm, out_hbm, idx_v, ones_v, zbuf, acc):
    c, s, w = _ids()

    def _fill_ones(r, carry):
        ones_v[r, :] = _ONE16
        return carry

    lax.fori_loop(0, CH, _fill_ones, 0)

    def _fill_zero(r, carry):
        zbuf[r, :] = _ZERO16
        return carry

    lax.fori_loop(0, ZCH, _fill_zero, 0)
    for j in range(RPS // ZCH):
        pltpu.sync_copy(zbuf, acc.at[pl.ds(s * RPS + j * ZCH, ZCH)])
    plsc.subcore_barrier()

    def _body(k, carry):
        base = w * EPW + k * CH
        pltpu.sync_copy(dst_hbm.at[pl.ds(base, CH)], idx_v)
        pltpu.sync_copy(ones_v, acc.at[idx_v], add=True)
        return carry

    lax.fori_loop(0, NCHUNK, _body, 0)
    plsc.subcore_barrier()
    pltpu.sync_copy(
        acc.at[pl.ds(s * RPS, RPS)], out_hbm.at[pl.ds(c * N + s * RPS, RPS)]
    )


@functools.partial(
    pl.kernel,
    out_type=jax.ShapeDtypeStruct((NC * N, D), jnp.float32),
    mesh=_mesh,
    scratch_types=[
        pltpu.VMEM((CH,), jnp.int32),
        pltpu.VMEM((CH,), jnp.int32),
        pltpu.VMEM((CH, D), jnp.float32),
        pltpu.VMEM((ZCH, D), jnp.float32),
        pltpu.VMEM_SHARED((N, D), jnp.float32),
        pltpu.SemaphoreType.DMA,
    ],
)
def _edge_agg(src_hbm, dst_hbm, g_hbm, out_hbm, sidx, didx, rows, zbuf, acc, sem):
    c, s, w = _ids()

    def _fill_zero(r, carry):
        for j in range(D // 16):
            zbuf[r, pl.ds(j * 16, 16)] = _ZERO16
        return carry

    lax.fori_loop(0, ZCH, _fill_zero, 0)
    for j in range(RPS // ZCH):
        pltpu.sync_copy(zbuf, acc.at[pl.ds(s * RPS + j * ZCH, ZCH)])
    plsc.subcore_barrier()

    def _body(k, carry):
        base = w * EPW + k * CH
        pltpu.sync_copy(src_hbm.at[pl.ds(base, CH)], sidx)
        pltpu.sync_copy(dst_hbm.at[pl.ds(base, CH)], didx)
        pltpu.async_copy(g_hbm.at[sidx], rows, sem).wait()
        pltpu.sync_copy(rows, acc.at[didx], add=True)
        return carry

    lax.fori_loop(0, NCHUNK, _body, 0)
    plsc.subcore_barrier()
    pltpu.sync_copy(
        acc.at[pl.ds(s * RPS, RPS)], out_hbm.at[pl.ds(c * N + s * RPS, RPS)]
    )


BN = 1000
GRID = N // BN


def _dis_of(p0v, p1v):
    return lax.rsqrt(1.0 + p0v[:, 0:1] + p1v[:, 0:1])


def _tc1_body(p0, p1, x, w1, g1):
    dis = _dis_of(p0[...], p1[...])
    h = jnp.dot(x[...], w1[...], preferred_element_type=jnp.float32)
    g1[...] = h * dis


def _tc2_body(p0, p1, a0, a1, g1, b1, w2, g2):
    dis = _dis_of(p0[...], p1[...])
    t = jnp.maximum(dis * (a0[...] + a1[...] + g1[...]) + b1[...], 0.0)
    h2 = jnp.dot(t, w2[...], preferred_element_type=jnp.float32)
    g2[...] = h2 * dis


def _tc3_body(p0, p1, a0, a1, g2, b2, out):
    dis = _dis_of(p0[...], p1[...])
    out[...] = dis * (a0[...] + a1[...] + g2[...]) + b2[...]


def _row_spec(w):
    return pl.BlockSpec((BN, w), lambda i: (i, 0))


def _full_spec(shape):
    return pl.BlockSpec(shape, lambda i: (0,) * len(shape))


_out_f32 = jax.ShapeDtypeStruct((N, D), jnp.float32)

_tc1 = pl.pallas_call(
    _tc1_body,
    grid=(GRID,),
    in_specs=[_row_spec(DEGW), _row_spec(DEGW), _row_spec(D), _full_spec((D, D))],
    out_specs=_row_spec(D),
    out_shape=_out_f32,
)

_tc2 = pl.pallas_call(
    _tc2_body,
    grid=(GRID,),
    in_specs=[
        _row_spec(DEGW),
        _row_spec(DEGW),
        _row_spec(D),
        _row_spec(D),
        _row_spec(D),
        _full_spec((1, D)),
        _full_spec((D, D)),
    ],
    out_specs=_row_spec(D),
    out_shape=_out_f32,
)

_tc3 = pl.pallas_call(
    _tc3_body,
    grid=(GRID,),
    in_specs=[
        _row_spec(DEGW),
        _row_spec(DEGW),
        _row_spec(D),
        _row_spec(D),
        _row_spec(D),
        _full_spec((1, D)),
    ],
    out_specs=_row_spec(D),
    out_shape=_out_f32,
)


def kernel(x, edge_index, W1, b1, W2, b2):
    src = edge_index[0]
    dst = edge_index[1]
    degp = _deg_hist(dst)
    p0, p1 = degp[:N], degp[N:]
    g1 = _tc1(p0, p1, x, W1)
    agg1 = _edge_agg(src, dst, g1)
    g2 = _tc2(p0, p1, agg1[:N], agg1[N:], g1, b1.reshape(1, D), W2)
    agg2 = _edge_agg(src, dst, g2)
    out = _tc3(p0, p1, agg2[:N], agg2[N:], g2, b2.reshape(1, D))
    return out


# speedup vs baseline: 13.0585x; 13.0585x over previous
import functools

import jax
import jax.numpy as jnp
from jax import lax
from jax.experimental import pallas as pl
from jax.experimental.pallas import tpu as pltpu
from jax.experimental.pallas import tpu_sc as plsc

N = 10000
E = 320000
D = 128
NC = 2
NS = 16
NW = NC * NS
EPW = E // NW
CH = 80
NCHUNK = EPW // CH
RPS = N // NS
ZCH = 125
DEGW = 16
OC = 632

_mesh = plsc.VectorSubcoreMesh(
    core_axis_name="c", subcore_axis_name="s", num_cores=NC, num_subcores=NS
)

def _ids():
    c = lax.axis_index("c")
    s = lax.axis_index("s")
    return c, s, c * NS + s


def _writeback(acc, out_hbm, c, s):
    tail = N - (NS - 1) * OC

    @pl.when(s < NS - 1)
    def _():
        pltpu.sync_copy(
            acc.at[pl.ds(s * OC, OC)], out_hbm.at[pl.ds(c * N + s * OC, OC)]
        )

    @pl.when(s == NS - 1)
    def _():
        pltpu.sync_copy(
            acc.at[pl.ds((NS - 1) * OC, tail)],
            out_hbm.at[pl.ds(c * N + (NS - 1) * OC, tail)],
        )


@functools.partial(
    pl.kernel,
    out_type=jax.ShapeDtypeStruct((NC * N, DEGW), jnp.float32),
    mesh=_mesh,
    scratch_types=[
        pltpu.VMEM((CH,), jnp.int32),
        pltpu.VMEM((CH, DEGW), jnp.float32),
        pltpu.VMEM((ZCH, DEGW), jnp.float32),
        pltpu.VMEM_SHARED((N, DEGW), jnp.float32),
    ],
)
def _deg_hist(dst_hbm, out_hbm, idx_v, ones_v, zbuf, acc):
    c, s, w = _ids()

    def _fill_ones(r, carry):
        ones_v[r, :] = jnp.ones((16,), jnp.float32)
        return carry

    lax.fori_loop(0, CH, _fill_ones, 0)

    def _fill_zero(r, carry):
        zbuf[r, :] = jnp.zeros((16,), jnp.float32)
        return carry

    lax.fori_loop(0, ZCH, _fill_zero, 0)
    for j in range(RPS // ZCH):
        pltpu.sync_copy(zbuf, acc.at[pl.ds(s * RPS + j * ZCH, ZCH)])
    plsc.subcore_barrier()

    def _body(k, carry):
        base = w * EPW + k * CH
        pltpu.sync_copy(dst_hbm.at[pl.ds(base, CH)], idx_v)
        pltpu.sync_copy(ones_v, acc.at[idx_v], add=True)
        return carry

    lax.fori_loop(0, NCHUNK, _body, 0)
    plsc.subcore_barrier()
    _writeback(acc, out_hbm, c, s)


@functools.partial(
    pl.kernel,
    out_type=jax.ShapeDtypeStruct((NC * N, D), jnp.float32),
    mesh=_mesh,
    scratch_types=[
        pltpu.VMEM((CH,), jnp.int32),
        pltpu.VMEM((CH,), jnp.int32),
        pltpu.VMEM((CH, D), jnp.float32),
        pltpu.VMEM((ZCH, D), jnp.float32),
        pltpu.VMEM_SHARED((N, D), jnp.float32),
        pltpu.SemaphoreType.DMA,
    ],
)
def _edge_agg(src_hbm, dst_hbm, g_hbm, out_hbm, sidx, didx, rows, zbuf, acc, sem):
    c, s, w = _ids()

    def _fill_zero(r, carry):
        for j in range(D // 16):
            zbuf[r, pl.ds(j * 16, 16)] = jnp.zeros((16,), jnp.float32)
        return carry

    lax.fori_loop(0, ZCH, _fill_zero, 0)
    for j in range(RPS // ZCH):
        pltpu.sync_copy(zbuf, acc.at[pl.ds(s * RPS + j * ZCH, ZCH)])
    plsc.subcore_barrier()

    def _body(k, carry):
        base = w * EPW + k * CH
        pltpu.sync_copy(src_hbm.at[pl.ds(base, CH)], sidx)
        pltpu.sync_copy(dst_hbm.at[pl.ds(base, CH)], didx)
        pltpu.async_copy(g_hbm.at[sidx], rows, sem).wait()
        pltpu.sync_copy(rows, acc.at[didx], add=True)
        return carry

    lax.fori_loop(0, NCHUNK, _body, 0)
    plsc.subcore_barrier()
    _writeback(acc, out_hbm, c, s)


BN = 1000
GRID = N // BN


def _dis_of(p0v, p1v):
    return lax.rsqrt(1.0 + p0v[:, 0:1] + p1v[:, 0:1])


def _tc1_body(p0, p1, x, w1, g1):
    dis = _dis_of(p0[...], p1[...])
    h = jnp.dot(x[...], w1[...], preferred_element_type=jnp.float32)
    g1[...] = h * dis


def _tc2_body(p0, p1, a0, a1, g1, b1, w2, g2):
    dis = _dis_of(p0[...], p1[...])
    t = jnp.maximum(dis * (a0[...] + a1[...] + g1[...]) + b1[...], 0.0)
    h2 = jnp.dot(t, w2[...], preferred_element_type=jnp.float32)
    g2[...] = h2 * dis


def _tc3_body(p0, p1, a0, a1, g2, b2, out):
    dis = _dis_of(p0[...], p1[...])
    out[...] = dis * (a0[...] + a1[...] + g2[...]) + b2[...]


def _row_spec(w):
    return pl.BlockSpec((BN, w), lambda i: (i, 0))


def _full_spec(shape):
    return pl.BlockSpec(shape, lambda i: (0,) * len(shape))


_out_f32 = jax.ShapeDtypeStruct((N, D), jnp.float32)

_tc1 = pl.pallas_call(
    _tc1_body,
    grid=(GRID,),
    in_specs=[_row_spec(DEGW), _row_spec(DEGW), _row_spec(D), _full_spec((D, D))],
    out_specs=_row_spec(D),
    out_shape=_out_f32,
)

_tc2 = pl.pallas_call(
    _tc2_body,
    grid=(GRID,),
    in_specs=[
        _row_spec(DEGW),
        _row_spec(DEGW),
        _row_spec(D),
        _row_spec(D),
        _row_spec(D),
        _full_spec((1, D)),
        _full_spec((D, D)),
    ],
    out_specs=_row_spec(D),
    out_shape=_out_f32,
)

_tc3 = pl.pallas_call(
    _tc3_body,
    grid=(GRID,),
    in_specs=[
        _row_spec(DEGW),
        _row_spec(DEGW),
        _row_spec(D),
        _row_spec(D),
        _row_spec(D),
        _full_spec((1, D)),
    ],
    out_specs=_row_spec(D),
    out_shape=_out_f32,
)


def kernel(x, edge_index, W1, b1, W2, b2):
    src = edge_index[0]
    dst = edge_index[1]
    degp = _deg_hist(dst)
    p0, p1 = degp[:N], degp[N:]
    g1 = _tc1(p0, p1, x, W1)
    agg1 = _edge_agg(src, dst, g1)
    g2 = _tc2(p0, p1, agg1[:N], agg1[N:], g1, b1.reshape(1, D), W2)
    agg2 = _edge_agg(src, dst, g2)
    out = _tc3(p0, p1, agg2[:N], agg2[N:], g2, b2.reshape(1, D))
    return out


# speedup vs baseline: 28.8668x; 2.2106x over previous
import functools

import jax
import jax.numpy as jnp
from jax import lax
from jax.experimental import pallas as pl
from jax.experimental.pallas import tpu as pltpu
from jax.experimental.pallas import tpu_sc as plsc

N = 10000
E = 320000
D = 128
NC = 2
NS = 16
NW = NC * NS
EPW = E // NW
CH = 80
NCHUNK = EPW // CH
NPASS = 5
IH = NCHUNK // NPASS
RPS = N // NS
ZCH = 125
DEGW = 16
OC = 632

_mesh = plsc.VectorSubcoreMesh(
    core_axis_name="c", subcore_axis_name="s", num_cores=NC, num_subcores=NS
)

def _ids():
    c = lax.axis_index("c")
    s = lax.axis_index("s")
    return c, s, c * NS + s


def _writeback(acc, out_hbm, c, s):
    tail = N - (NS - 1) * OC

    @pl.when(s < NS - 1)
    def _():
        pltpu.sync_copy(
            acc.at[pl.ds(s * OC, OC)], out_hbm.at[pl.ds(c * N + s * OC, OC)]
        )

    @pl.when(s == NS - 1)
    def _():
        pltpu.sync_copy(
            acc.at[pl.ds((NS - 1) * OC, tail)],
            out_hbm.at[pl.ds(c * N + (NS - 1) * OC, tail)],
        )


@functools.partial(
    pl.kernel,
    out_type=jax.ShapeDtypeStruct((NC * N, DEGW), jnp.float32),
    mesh=_mesh,
    scratch_types=[
        pltpu.VMEM((NCHUNK, CH), jnp.int32),
        pltpu.VMEM((CH, DEGW), jnp.float32),
        pltpu.VMEM((ZCH, DEGW), jnp.float32),
        pltpu.VMEM_SHARED((N, DEGW), jnp.float32),
        pltpu.SemaphoreType.DMA,
    ],
)
def _deg_hist(dst_hbm, out_hbm, didx, ones_v, zbuf, acc, sem):
    c, s, w = _ids()

    def _fill_ones(r, carry):
        ones_v[r, :] = jnp.ones((16,), jnp.float32)
        return carry

    lax.fori_loop(0, CH, _fill_ones, 0)

    def _fill_zero(r, carry):
        zbuf[r, :] = jnp.zeros((16,), jnp.float32)
        return carry

    lax.fori_loop(0, ZCH, _fill_zero, 0)
    for j in range(RPS // ZCH):
        pltpu.sync_copy(zbuf, acc.at[pl.ds(s * RPS + j * ZCH, ZCH)])
    for p in range(NPASS):
        pltpu.sync_copy(dst_hbm.at[w, p], didx.at[pl.ds(p * IH, IH)])
    plsc.subcore_barrier()

    def _fire(k, carry):
        pltpu.async_copy(ones_v, acc.at[didx.at[k]], sem, add=True)
        return carry

    lax.fori_loop(0, NCHUNK, _fire, 0)

    def _drain(k, carry):
        pltpu.make_async_copy(ones_v, acc.at[didx.at[k]], sem).wait()
        return carry

    lax.fori_loop(0, NCHUNK, _drain, 0)
    plsc.subcore_barrier()
    _writeback(acc, out_hbm, c, s)


@functools.partial(
    pl.kernel,
    out_type=jax.ShapeDtypeStruct((NC * N, D), jnp.float32),
    mesh=_mesh,
    scratch_types=[
        pltpu.VMEM((2, IH, CH), jnp.int32),
        pltpu.VMEM((2, IH, CH), jnp.int32),
        pltpu.VMEM((CH, D), jnp.float32),
        pltpu.VMEM((CH, D), jnp.float32),
        pltpu.VMEM_SHARED((N, D), jnp.float32),
        pltpu.SemaphoreType.DMA,
        pltpu.SemaphoreType.DMA,
        pltpu.SemaphoreType.DMA,
    ],
)
def _edge_agg(
    src_hbm, dst_hbm, g_hbm, out_hbm, sidx, didx, rows0, rows1, acc, s0, s1, si
):
    c, s, w = _ids()

    def _fill_zero(r, carry):
        for j in range(D // 16):
            rows0[r, pl.ds(j * 16, 16)] = jnp.zeros((16,), jnp.float32)
        return carry

    lax.fori_loop(0, CH, _fill_zero, 0)
    for j in range(RPS // CH):
        pltpu.sync_copy(rows0, acc.at[pl.ds(s * RPS + j * CH, CH)])
    pltpu.sync_copy(
        rows0.at[pl.ds(0, RPS - (RPS // CH) * CH)],
        acc.at[pl.ds(s * RPS + (RPS // CH) * CH, RPS - (RPS // CH) * CH)],
    )

    pltpu.async_copy(src_hbm.at[w, 0], sidx.at[0], si)
    pltpu.async_copy(dst_hbm.at[w, 0], didx.at[0], si)
    plsc.subcore_barrier()

    for p in range(NPASS):
        pb = p % 2
        pltpu.make_async_copy(src_hbm.at[w, p], sidx.at[pb], si).wait()
        pltpu.make_async_copy(dst_hbm.at[w, p], didx.at[pb], si).wait()
        if p + 1 < NPASS:
            pltpu.async_copy(src_hbm.at[w, p + 1], sidx.at[1 - pb], si)
            pltpu.async_copy(dst_hbm.at[w, p + 1], didx.at[1 - pb], si)
        sl = sidx.at[pb]
        dl = didx.at[pb]

        pltpu.async_copy(g_hbm.at[sl.at[0]], rows0, s0)

        def _body(j, carry, sl=sl, dl=dl):
            k0 = 2 * j
            pltpu.async_copy(g_hbm.at[sl.at[k0 + 1]], rows1, s1)
            pltpu.make_async_copy(g_hbm.at[sl.at[k0]], rows0, s0).wait()
            pltpu.sync_copy(rows0, acc.at[dl.at[k0]], add=True)
            pltpu.async_copy(g_hbm.at[sl.at[k0 + 2]], rows0, s0)
            pltpu.make_async_copy(g_hbm.at[sl.at[k0 + 1]], rows1, s1).wait()
            pltpu.sync_copy(rows1, acc.at[dl.at[k0 + 1]], add=True)
            return carry

        lax.fori_loop(0, (IH - 1) // 2, _body, 0)
        pltpu.make_async_copy(g_hbm.at[sl.at[IH - 1]], rows0, s0).wait()
        pltpu.sync_copy(rows0, acc.at[dl.at[IH - 1]], add=True)

    plsc.subcore_barrier()
    _writeback(acc, out_hbm, c, s)


BN = 1000
GRID = N // BN


def _dis_of(p0v, p1v):
    return lax.rsqrt(1.0 + p0v[:, 0:1] + p1v[:, 0:1])


def _tc1_body(p0, p1, x, w1, g1):
    dis = _dis_of(p0[...], p1[...])
    h = jnp.dot(x[...], w1[...], preferred_element_type=jnp.float32)
    g1[...] = h * dis


def _tc2_body(p0, p1, a0, a1, g1, b1, w2, g2):
    dis = _dis_of(p0[...], p1[...])
    t = jnp.maximum(dis * (a0[...] + a1[...] + g1[...]) + b1[...], 0.0)
    h2 = jnp.dot(t, w2[...], preferred_element_type=jnp.float32)
    g2[...] = h2 * dis


def _tc3_body(p0, p1, a0, a1, g2, b2, out):
    dis = _dis_of(p0[...], p1[...])
    out[...] = dis * (a0[...] + a1[...] + g2[...]) + b2[...]


def _row_spec(w):
    return pl.BlockSpec((BN, w), lambda i: (i, 0))


def _full_spec(shape):
    return pl.BlockSpec(shape, lambda i: (0,) * len(shape))


_out_f32 = jax.ShapeDtypeStruct((N, D), jnp.float32)

_tc1 = pl.pallas_call(
    _tc1_body,
    grid=(GRID,),
    in_specs=[_row_spec(DEGW), _row_spec(DEGW), _row_spec(D), _full_spec((D, D))],
    out_specs=_row_spec(D),
    out_shape=_out_f32,
)

_tc2 = pl.pallas_call(
    _tc2_body,
    grid=(GRID,),
    in_specs=[
        _row_spec(DEGW),
        _row_spec(DEGW),
        _row_spec(D),
        _row_spec(D),
        _row_spec(D),
        _full_spec((1, D)),
        _full_spec((D, D)),
    ],
    out_specs=_row_spec(D),
    out_shape=_out_f32,
)

_tc3 = pl.pallas_call(
    _tc3_body,
    grid=(GRID,),
    in_specs=[
        _row_spec(DEGW),
        _row_spec(DEGW),
        _row_spec(D),
        _row_spec(D),
        _row_spec(D),
        _full_spec((1, D)),
    ],
    out_specs=_row_spec(D),
    out_shape=_out_f32,
)


def kernel(x, edge_index, W1, b1, W2, b2):
    src = edge_index[0].reshape(NW, NPASS, IH, CH)
    dst = edge_index[1].reshape(NW, NPASS, IH, CH)
    degp = _deg_hist(dst)
    p0, p1 = degp[:N], degp[N:]
    g1 = _tc1(p0, p1, x, W1)
    agg1 = _edge_agg(src, dst, g1)
    g2 = _tc2(p0, p1, agg1[:N], agg1[N:], g1, b1.reshape(1, D), W2)
    agg2 = _edge_agg(src, dst, g2)
    out = _tc3(p0, p1, agg2[:N], agg2[N:], g2, b2.reshape(1, D))
    return out


# speedup vs baseline: 30.8534x; 1.0688x over previous
import functools

import jax
import jax.numpy as jnp
from jax import lax
from jax.experimental import pallas as pl
from jax.experimental.pallas import tpu as pltpu
from jax.experimental.pallas import tpu_sc as plsc

N = 10000
E = 320000
D = 128
NC = 2
NS = 16
NW = NC * NS
EPW = E // NW
CH = 80
NCHUNK = EPW // CH
NPASS = 5
IH = NCHUNK // NPASS
RPS = N // NS
ZCH = 125
DEGW = 16
OC = 632

_mesh = plsc.VectorSubcoreMesh(
    core_axis_name="c", subcore_axis_name="s", num_cores=NC, num_subcores=NS
)

def _ids():
    c = lax.axis_index("c")
    s = lax.axis_index("s")
    return c, s, c * NS + s


def _writeback(acc, out0_hbm, out1_hbm, c, s):
    tail = N - (NS - 1) * OC

    def _copy(out_hbm):
        @pl.when(s < NS - 1)
        def _():
            pltpu.sync_copy(
                acc.at[pl.ds(s * OC, OC)], out_hbm.at[pl.ds(s * OC, OC)]
            )

        @pl.when(s == NS - 1)
        def _():
            pltpu.sync_copy(
                acc.at[pl.ds((NS - 1) * OC, tail)],
                out_hbm.at[pl.ds((NS - 1) * OC, tail)],
            )

    @pl.when(c == 0)
    def _():
        _copy(out0_hbm)

    @pl.when(c == 1)
    def _():
        _copy(out1_hbm)


@functools.partial(
    pl.kernel,
    out_type=(
        jax.ShapeDtypeStruct((N, DEGW), jnp.float32),
        jax.ShapeDtypeStruct((N, DEGW), jnp.float32),
    ),
    mesh=_mesh,
    scratch_types=[
        pltpu.VMEM((NCHUNK, CH), jnp.int32),
        pltpu.VMEM((CH, DEGW), jnp.float32),
        pltpu.VMEM((ZCH, DEGW), jnp.float32),
        pltpu.VMEM_SHARED((N, DEGW), jnp.float32),
        pltpu.SemaphoreType.DMA,
        pltpu.SemaphoreType.DMA,
    ],
)
def _deg_hist(dst_hbm, out0_hbm, out1_hbm, didx, ones_v, zbuf, acc, sem, si):
    c, s, w = _ids()

    def _fill_ones(r, carry):
        ones_v[r, :] = jnp.ones((16,), jnp.float32)
        return carry

    lax.fori_loop(0, CH, _fill_ones, 0)

    def _fill_zero(r, carry):
        zbuf[r, :] = jnp.zeros((16,), jnp.float32)
        return carry

    def _stage(k, carry):
        pltpu.async_copy(dst_hbm.at[pl.ds(w * EPW + k * CH, CH)], didx.at[k], si)
        return carry

    lax.fori_loop(0, NCHUNK, _stage, 0)

    lax.fori_loop(0, ZCH, _fill_zero, 0)
    for j in range(RPS // ZCH):
        pltpu.sync_copy(zbuf, acc.at[pl.ds(s * RPS + j * ZCH, ZCH)])

    def _stage_wait(k, carry):
        pltpu.make_async_copy(
            dst_hbm.at[pl.ds(w * EPW + k * CH, CH)], didx.at[k], si
        ).wait()
        return carry

    lax.fori_loop(0, NCHUNK, _stage_wait, 0)
    plsc.subcore_barrier()

    def _fire(k, carry):
        pltpu.async_copy(ones_v, acc.at[didx.at[k]], sem, add=True)
        return carry

    lax.fori_loop(0, NCHUNK, _fire, 0)

    def _drain(k, carry):
        pltpu.make_async_copy(ones_v, acc.at[didx.at[k]], sem).wait()
        return carry

    lax.fori_loop(0, NCHUNK, _drain, 0)
    plsc.subcore_barrier()
    _writeback(acc, out0_hbm, out1_hbm, c, s)


SLAB = IH * CH


@functools.partial(
    pl.kernel,
    out_type=(
        jax.ShapeDtypeStruct((N, D), jnp.float32),
        jax.ShapeDtypeStruct((N, D), jnp.float32),
    ),
    mesh=_mesh,
    scratch_types=[
        pltpu.VMEM((2, IH, CH), jnp.int32),
        pltpu.VMEM((2, IH, CH), jnp.int32),
        pltpu.VMEM((CH, D), jnp.float32),
        pltpu.VMEM((CH, D), jnp.float32),
        pltpu.VMEM_SHARED((N, D), jnp.float32),
        pltpu.SemaphoreType.DMA,
        pltpu.SemaphoreType.DMA,
        pltpu.SemaphoreType.DMA,
    ],
)
def _edge_agg(
    src_hbm, dst_hbm, g_hbm, out0_hbm, out1_hbm,
    sidx, didx, rows0, rows1, acc, s0, s1, si,
):
    c, s, w = _ids()

    def _fill_zero(r, carry):
        for j in range(D // 16):
            rows0[r, pl.ds(j * 16, 16)] = jnp.zeros((16,), jnp.float32)
        return carry

    lax.fori_loop(0, CH, _fill_zero, 0)
    for j in range(RPS // CH):
        pltpu.sync_copy(rows0, acc.at[pl.ds(s * RPS + j * CH, CH)])
    pltpu.sync_copy(
        rows0.at[pl.ds(0, RPS - (RPS // CH) * CH)],
        acc.at[pl.ds(s * RPS + (RPS // CH) * CH, RPS - (RPS // CH) * CH)],
    )

    def _stage(p, pb):
        def _rows(r, carry):
            base = w * EPW + p * SLAB + r * CH
            pltpu.async_copy(src_hbm.at[pl.ds(base, CH)], sidx.at[pb, r], si)
            pltpu.async_copy(dst_hbm.at[pl.ds(base, CH)], didx.at[pb, r], si)
            return carry

        lax.fori_loop(0, IH, _rows, 0)

    def _stage_wait(p, pb):
        def _rows(r, carry):
            base = w * EPW + p * SLAB + r * CH
            pltpu.make_async_copy(
                src_hbm.at[pl.ds(base, CH)], sidx.at[pb, r], si
            ).wait()
            pltpu.make_async_copy(
                dst_hbm.at[pl.ds(base, CH)], didx.at[pb, r], si
            ).wait()
            return carry

        lax.fori_loop(0, IH, _rows, 0)

    _stage(0, 0)
    plsc.subcore_barrier()

    for p in range(NPASS):
        pb = p % 2
        _stage_wait(p, pb)
        if p + 1 < NPASS:
            _stage(p + 1, 1 - pb)
        sl = sidx.at[pb]
        dl = didx.at[pb]

        pltpu.async_copy(g_hbm.at[sl.at[0]], rows0, s0)

        def _body(j, carry, sl=sl, dl=dl):
            k0 = 2 * j
            pltpu.async_copy(g_hbm.at[sl.at[k0 + 1]], rows1, s1)
            pltpu.make_async_copy(g_hbm.at[sl.at[k0]], rows0, s0).wait()
            pltpu.sync_copy(rows0, acc.at[dl.at[k0]], add=True)
            pltpu.async_copy(g_hbm.at[sl.at[k0 + 2]], rows0, s0)
            pltpu.make_async_copy(g_hbm.at[sl.at[k0 + 1]], rows1, s1).wait()
            pltpu.sync_copy(rows1, acc.at[dl.at[k0 + 1]], add=True)
            return carry

        lax.fori_loop(0, (IH - 1) // 2, _body, 0)
        pltpu.make_async_copy(g_hbm.at[sl.at[IH - 1]], rows0, s0).wait()
        pltpu.sync_copy(rows0, acc.at[dl.at[IH - 1]], add=True)

    plsc.subcore_barrier()
    _writeback(acc, out0_hbm, out1_hbm, c, s)


BN = 1000
GRID = N // BN


def _dis_of(p0v, p1v):
    return lax.rsqrt(1.0 + p0v[:, 0:1] + p1v[:, 0:1])


def _tc1_body(p0, p1, x, w1, g1):
    dis = _dis_of(p0[...], p1[...])
    h = jnp.dot(x[...], w1[...], preferred_element_type=jnp.float32)
    g1[...] = h * dis


def _tc2_body(p0, p1, a0, a1, g1, b1, w2, g2):
    dis = _dis_of(p0[...], p1[...])
    t = jnp.maximum(dis * (a0[...] + a1[...] + g1[...]) + b1[...], 0.0)
    h2 = jnp.dot(t, w2[...], preferred_element_type=jnp.float32)
    g2[...] = h2 * dis


def _tc3_body(p0, p1, a0, a1, g2, b2, out):
    dis = _dis_of(p0[...], p1[...])
    out[...] = dis * (a0[...] + a1[...] + g2[...]) + b2[...]


def _row_spec(w):
    return pl.BlockSpec((BN, w), lambda i: (i, 0))


def _full_spec(shape):
    return pl.BlockSpec(shape, lambda i: (0,) * len(shape))


_out_f32 = jax.ShapeDtypeStruct((N, D), jnp.float32)

_tc1 = pl.pallas_call(
    _tc1_body,
    grid=(GRID,),
    in_specs=[_row_spec(DEGW), _row_spec(DEGW), _row_spec(D), _full_spec((D, D))],
    out_specs=_row_spec(D),
    out_shape=_out_f32,
)

_tc2 = pl.pallas_call(
    _tc2_body,
    grid=(GRID,),
    in_specs=[
        _row_spec(DEGW),
        _row_spec(DEGW),
        _row_spec(D),
        _row_spec(D),
        _row_spec(D),
        _full_spec((1, D)),
        _full_spec((D, D)),
    ],
    out_specs=_row_spec(D),
    out_shape=_out_f32,
)

_tc3 = pl.pallas_call(
    _tc3_body,
    grid=(GRID,),
    in_specs=[
        _row_spec(DEGW),
        _row_spec(DEGW),
        _row_spec(D),
        _row_spec(D),
        _row_spec(D),
        _full_spec((1, D)),
    ],
    out_specs=_row_spec(D),
    out_shape=_out_f32,
)


def kernel(x, edge_index, W1, b1, W2, b2):
    src = edge_index[0]
    dst = edge_index[1]
    p0, p1 = _deg_hist(dst)
    g1 = _tc1(p0, p1, x, W1)
    a10, a11 = _edge_agg(src, dst, g1)
    g2 = _tc2(p0, p1, a10, a11, g1, b1.reshape(1, D), W2)
    a20, a21 = _edge_agg(src, dst, g2)
    out = _tc3(p0, p1, a20, a21, g2, b2.reshape(1, D))
    return out
